# baseline (speedup 1.0000x reference)
import functools

import jax
import jax.numpy as jnp
from jax import lax
from jax.experimental import pallas as pl
from jax.experimental.pallas import tpu as pltpu
from jax.experimental.pallas import tpu_sc as plsc

B, C, H, W = 4, 192, 224, 224
P = H * W
NC, NS, L = 2, 16, 16
NW = NC * NS
PLANES = B * C
PLANES_PER_TILE = PLANES // NW
PAIRS_PER_TILE = PLANES_PER_TILE // 2
CHUNK = 1568
NCHUNK = P // CHUNK
UNROLL = 8
MASKL = jnp.int32(0xFFFF)
MASKH = jnp.int32(-0x10000)


def _coef_body(xs_ref, ys_ref, pk):
    xs = xs_ref[...]
    ys = ys_ref[...]
    x = (xs + 1.0) * jnp.float32(W) / 2.0
    y = (ys + 1.0) * jnp.float32(H) / 2.0
    x0r = jnp.floor(x).astype(jnp.int32)
    y0r = jnp.floor(y).astype(jnp.int32)
    x0 = jnp.clip(x0r, 0, W - 1)
    x1 = jnp.clip(x0r + 1, 0, W - 1)
    y0 = jnp.clip(y0r, 0, H - 1)
    y1 = jnp.clip(y0r + 1, 0, H - 1)
    x0f = x0.astype(jnp.float32)
    x1f = x1.astype(jnp.float32)
    y0f = y0.astype(jnp.float32)
    y1f = y1.astype(jnp.float32)
    ia = y0 * W + x0
    ib = y1 * W + x0
    ic = y0 * W + x1
    idd = y1 * W + x1

    def b16(v):
        return lax.bitcast_convert_type(
            v.astype(jnp.bfloat16), jnp.uint16).astype(jnp.int32)

    pk[:, :, 0, :] = ia | (ic << 16)
    pk[:, :, 1, :] = ib | (idd << 16)
    pk[:, :, 2, :] = (b16(x1f - x) << 16) | b16(x - x0f)
    pk[:, :, 3, :] = (b16(y1f - y) << 16) | b16(y - y0f)


def _coefs(theta):
    x_lin = jnp.linspace(-1.0, 1.0, W, dtype=jnp.float32)
    y_lin = jnp.linspace(-1.0, 1.0, H, dtype=jnp.float32)
    x_t = jnp.tile(x_lin[None, :], (H, 1))
    y_t = jnp.tile(y_lin[:, None], (1, W))
    grid = jnp.stack(
        [x_t.ravel(), y_t.ravel(), jnp.ones(P, dtype=jnp.float32)], axis=0)
    theta_r = theta.reshape(-1, 2, 3)
    t_g = jnp.einsum('bij,jp->bip', theta_r, grid)
    xs = t_g[:, 0, :].reshape(B, NCHUNK, CHUNK)
    ys = t_g[:, 1, :].reshape(B, NCHUNK, CHUNK)
    return pl.pallas_call(
        _coef_body,
        out_shape=jax.ShapeDtypeStruct((B, NCHUNK, 4, CHUNK), jnp.int32),
    )(xs, ys)


def _sc_body(img_hbm, pk_hbm, out_hbm,
             img0, img1, pk0, pk1, oa0, oa1, ob0, ob1,
             sem0, sem1, semw0, semw1):
    wid = lax.axis_index("s") * NC + lax.axis_index("c")
    base_g = wid * PLANES_PER_TILE
    b = base_g // C

    def fire(k, buf, sem):
        src = pk_hbm.at[pl.ds((b * NCHUNK + k) * (4 * CHUNK), 4 * CHUNK)]
        pltpu.async_copy(src, buf, sem)

    def drain(buf, sem):
        pltpu.make_async_copy(
            pk_hbm.at[pl.ds(0, 4 * CHUNK)], buf, sem).wait()

    def compute(buf, outa, outb):
        @plsc.parallel_loop(0, CHUNK, step=L, unroll=UNROLL)
        def _body(o):
            w1 = buf[pl.ds(0 * CHUNK + o, L)]
            w2 = buf[pl.ds(1 * CHUNK + o, L)]
            gxp = buf[pl.ds(2 * CHUNK + o, L)]
            gyp = buf[pl.ds(3 * CHUNK + o, L)]
            ia = w1 & MASKL
            ic = lax.shift_right_logical(w1, 16)
            ib = w2 & MASKL
            idd = lax.shift_right_logical(w2, 16)
            gx0 = plsc.bitcast(gxp & MASKH, jnp.float32)
            gx1 = plsc.bitcast(gxp << 16, jnp.float32)
            gy0 = plsc.bitcast(gyp & MASKH, jnp.float32)
            gy1 = plsc.bitcast(gyp << 16, jnp.float32)
            for imgv, outv in ((img0, outa), (img1, outb)):
                va = plsc.load_gather(imgv, [ia])
                vb = plsc.load_gather(imgv, [ib])
                vc = plsc.load_gather(imgv, [ic])
                vd = plsc.load_gather(imgv, [idd])
                outv[pl.ds(o, L)] = (gy0 * (gx0 * va + gx1 * vc)
                                     + gy1 * (gx0 * vb + gx1 * vd))

    def wout(g, k, outa, outb, semw):
        pltpu.async_copy(
            outa, out_hbm.at[pl.ds(g * P + k * CHUNK, CHUNK)], semw)
        pltpu.async_copy(
            outb, out_hbm.at[pl.ds((g + 1) * P + k * CHUNK, CHUNK)], semw)

    def wdrain(outa, outb, semw):
        pltpu.make_async_copy(
            img_hbm.at[pl.ds(0, CHUNK)], outa, semw).wait()
        pltpu.make_async_copy(
            img_hbm.at[pl.ds(0, CHUNK)], outb, semw).wait()

    def per_pair(pr, carry):
        g = base_g + 2 * pr
        pltpu.sync_copy(img_hbm.at[pl.ds(g * P, P)], img0)
        pltpu.sync_copy(img_hbm.at[pl.ds((g + 1) * P, P)], img1)
        fire(0, pk0, sem0)

        def pair(k2, carry2):
            k = 2 * k2
            fire(k + 1, pk1, sem1)
            drain(pk0, sem0)

            @pl.when(k2 > 0)
            def _():
                wdrain(oa0, ob0, semw0)

            compute(pk0, oa0, ob0)
            wout(g, k, oa0, ob0, semw0)

            @pl.when(k2 < NCHUNK // 2 - 1)
            def _():
                fire(k + 2, pk0, sem0)

            drain(pk1, sem1)

            @pl.when(k2 > 0)
            def _():
                wdrain(oa1, ob1, semw1)

            compute(pk1, oa1, ob1)
            wout(g, k + 1, oa1, ob1, semw1)
            return carry2

        lax.fori_loop(0, NCHUNK // 2, pair, jnp.int32(0))
        wdrain(oa0, ob0, semw0)
        wdrain(oa1, ob1, semw1)
        return carry

    lax.fori_loop(0, PAIRS_PER_TILE, per_pair, jnp.int32(0))


@functools.cache
def _sc_sample_call():
    return pl.kernel(
        _sc_body,
        out_type=jax.ShapeDtypeStruct((PLANES * P,), jnp.float32),
        mesh=plsc.VectorSubcoreMesh(
            core_axis_name="c", subcore_axis_name="s",
            num_cores=NC, num_subcores=NS),
        compiler_params=pltpu.CompilerParams(needs_layout_passes=False),
        scratch_types=[
            pltpu.VMEM((P,), jnp.float32),
            pltpu.VMEM((P,), jnp.float32),
            pltpu.VMEM((4 * CHUNK,), jnp.int32),
            pltpu.VMEM((4 * CHUNK,), jnp.int32),
            pltpu.VMEM((CHUNK,), jnp.float32),
            pltpu.VMEM((CHUNK,), jnp.float32),
            pltpu.VMEM((CHUNK,), jnp.float32),
            pltpu.VMEM((CHUNK,), jnp.float32),
            pltpu.SemaphoreType.DMA,
            pltpu.SemaphoreType.DMA,
            pltpu.SemaphoreType.DMA,
            pltpu.SemaphoreType.DMA,
        ],
    )


def kernel(conv_input, theta):
    pk = _coefs(theta).reshape(-1)
    img = conv_input.reshape(PLANES * P)
    out = _sc_sample_call()(img, pk)
    return out.reshape(B, C, H, W)


# speedup vs baseline: 1.0525x; 1.0525x over previous
import functools

import jax
import jax.numpy as jnp
from jax import lax
from jax.experimental import pallas as pl
from jax.experimental.pallas import tpu as pltpu
from jax.experimental.pallas import tpu_sc as plsc

B, C, H, W = 4, 192, 224, 224
P = H * W
NC, NS, L = 2, 16, 16
NW = NC * NS
PLANES = B * C
PLANES_PER_TILE = PLANES // NW
PAIRS_PER_TILE = PLANES_PER_TILE // 2
CHUNK = 1568
NCHUNK = P // CHUNK
UNROLL = 2
MASKL = jnp.int32(0xFFFF)
MASKH = jnp.int32(-0x10000)


def _coef_body(xs_ref, ys_ref, pk):
    xs = xs_ref[...]
    ys = ys_ref[...]
    x = (xs + 1.0) * jnp.float32(W) / 2.0
    y = (ys + 1.0) * jnp.float32(H) / 2.0
    x0r = jnp.floor(x).astype(jnp.int32)
    y0r = jnp.floor(y).astype(jnp.int32)
    x0 = jnp.clip(x0r, 0, W - 1)
    x1 = jnp.clip(x0r + 1, 0, W - 1)
    y0 = jnp.clip(y0r, 0, H - 1)
    y1 = jnp.clip(y0r + 1, 0, H - 1)
    x0f = x0.astype(jnp.float32)
    x1f = x1.astype(jnp.float32)
    y0f = y0.astype(jnp.float32)
    y1f = y1.astype(jnp.float32)
    ia = y0 * W + x0
    ib = y1 * W + x0
    ic = y0 * W + x1
    idd = y1 * W + x1

    def b16(v):
        return lax.bitcast_convert_type(
            v.astype(jnp.bfloat16), jnp.uint16).astype(jnp.int32)

    pk[:, :, 0, :] = ia | (ic << 16)
    pk[:, :, 1, :] = ib | (idd << 16)
    pk[:, :, 2, :] = (b16(x1f - x) << 16) | b16(x - x0f)
    pk[:, :, 3, :] = (b16(y1f - y) << 16) | b16(y - y0f)


def _coefs(theta):
    x_lin = jnp.linspace(-1.0, 1.0, W, dtype=jnp.float32)
    y_lin = jnp.linspace(-1.0, 1.0, H, dtype=jnp.float32)
    x_t = jnp.tile(x_lin[None, :], (H, 1))
    y_t = jnp.tile(y_lin[:, None], (1, W))
    grid = jnp.stack(
        [x_t.ravel(), y_t.ravel(), jnp.ones(P, dtype=jnp.float32)], axis=0)
    theta_r = theta.reshape(-1, 2, 3)
    t_g = jnp.einsum('bij,jp->bip', theta_r, grid)
    xs = t_g[:, 0, :].reshape(B, NCHUNK, CHUNK)
    ys = t_g[:, 1, :].reshape(B, NCHUNK, CHUNK)
    return pl.pallas_call(
        _coef_body,
        out_shape=jax.ShapeDtypeStruct((B, NCHUNK, 4, CHUNK), jnp.int32),
    )(xs, ys)


def _sc_body(img_hbm, pk_hbm, out_hbm,
             img0, img1, pk0, pk1, oa0, oa1, ob0, ob1,
             sem0, sem1, semw0, semw1):
    wid = lax.axis_index("s") * NC + lax.axis_index("c")
    base_g = wid * PLANES_PER_TILE
    b = base_g // C

    def fire(k, buf, sem):
        src = pk_hbm.at[pl.ds((b * NCHUNK + k) * (4 * CHUNK), 4 * CHUNK)]
        pltpu.async_copy(src, buf, sem)

    def drain(buf, sem):
        pltpu.make_async_copy(
            pk_hbm.at[pl.ds(0, 4 * CHUNK)], buf, sem).wait()

    def compute(buf, outa, outb):
        @plsc.parallel_loop(0, CHUNK, step=L, unroll=UNROLL)
        def _body(o):
            w1 = buf[pl.ds(0 * CHUNK + o, L)]
            w2 = buf[pl.ds(1 * CHUNK + o, L)]
            gxp = buf[pl.ds(2 * CHUNK + o, L)]
            gyp = buf[pl.ds(3 * CHUNK + o, L)]
            ia = w1 & MASKL
            ic = lax.shift_right_logical(w1, 16)
            ib = w2 & MASKL
            idd = lax.shift_right_logical(w2, 16)
            gx0 = plsc.bitcast(gxp & MASKH, jnp.float32)
            gx1 = plsc.bitcast(gxp << 16, jnp.float32)
            gy0 = plsc.bitcast(gyp & MASKH, jnp.float32)
            gy1 = plsc.bitcast(gyp << 16, jnp.float32)
            for imgv, outv in ((img0, outa), (img1, outb)):
                va = plsc.load_gather(imgv, [ia])
                vb = plsc.load_gather(imgv, [ib])
                vc = plsc.load_gather(imgv, [ic])
                vd = plsc.load_gather(imgv, [idd])
                outv[pl.ds(o, L)] = (gy0 * (gx0 * va + gx1 * vc)
                                     + gy1 * (gx0 * vb + gx1 * vd))

    def wout(g, k, outa, outb, semw):
        pltpu.async_copy(
            outa, out_hbm.at[pl.ds(g * P + k * CHUNK, CHUNK)], semw)
        pltpu.async_copy(
            outb, out_hbm.at[pl.ds((g + 1) * P + k * CHUNK, CHUNK)], semw)

    def wdrain(outa, outb, semw):
        pltpu.make_async_copy(
            img_hbm.at[pl.ds(0, CHUNK)], outa, semw).wait()
        pltpu.make_async_copy(
            img_hbm.at[pl.ds(0, CHUNK)], outb, semw).wait()

    def per_pair(pr, carry):
        g = base_g + 2 * pr
        pltpu.sync_copy(img_hbm.at[pl.ds(g * P, P)], img0)
        pltpu.sync_copy(img_hbm.at[pl.ds((g + 1) * P, P)], img1)
        fire(0, pk0, sem0)

        def pair(k2, carry2):
            k = 2 * k2
            fire(k + 1, pk1, sem1)
            drain(pk0, sem0)

            @pl.when(k2 > 0)
            def _():
                wdrain(oa0, ob0, semw0)

            compute(pk0, oa0, ob0)
            wout(g, k, oa0, ob0, semw0)

            @pl.when(k2 < NCHUNK // 2 - 1)
            def _():
                fire(k + 2, pk0, sem0)

            drain(pk1, sem1)

            @pl.when(k2 > 0)
            def _():
                wdrain(oa1, ob1, semw1)

            compute(pk1, oa1, ob1)
            wout(g, k + 1, oa1, ob1, semw1)
            return carry2

        lax.fori_loop(0, NCHUNK // 2, pair, jnp.int32(0))
        wdrain(oa0, ob0, semw0)
        wdrain(oa1, ob1, semw1)
        return carry

    lax.fori_loop(0, PAIRS_PER_TILE, per_pair, jnp.int32(0))


@functools.cache
def _sc_sample_call():
    return pl.kernel(
        _sc_body,
        out_type=jax.ShapeDtypeStruct((PLANES * P,), jnp.float32),
        mesh=plsc.VectorSubcoreMesh(
            core_axis_name="c", subcore_axis_name="s",
            num_cores=NC, num_subcores=NS),
        compiler_params=pltpu.CompilerParams(needs_layout_passes=False),
        scratch_types=[
            pltpu.VMEM((P,), jnp.float32),
            pltpu.VMEM((P,), jnp.float32),
            pltpu.VMEM((4 * CHUNK,), jnp.int32),
            pltpu.VMEM((4 * CHUNK,), jnp.int32),
            pltpu.VMEM((CHUNK,), jnp.float32),
            pltpu.VMEM((CHUNK,), jnp.float32),
            pltpu.VMEM((CHUNK,), jnp.float32),
            pltpu.VMEM((CHUNK,), jnp.float32),
            pltpu.SemaphoreType.DMA,
            pltpu.SemaphoreType.DMA,
            pltpu.SemaphoreType.DMA,
            pltpu.SemaphoreType.DMA,
        ],
    )


def kernel(conv_input, theta):
    pk = _coefs(theta).reshape(-1)
    img = conv_input.reshape(PLANES * P)
    out = _sc_sample_call()(img, pk)
    return out.reshape(B, C, H, W)


# speedup vs baseline: 1.2182x; 1.1574x over previous
import functools

import jax
import jax.numpy as jnp
from jax import lax
from jax.experimental import pallas as pl
from jax.experimental.pallas import tpu as pltpu
from jax.experimental.pallas import tpu_sc as plsc

B, C, H, W = 4, 192, 224, 224
P = H * W
NC, NS, L = 2, 16, 16
NW = NC * NS
PLANES = B * C
PLANES_PER_TILE = PLANES // NW
PAIRS_PER_TILE = PLANES_PER_TILE // 2
CHUNK = 1792
NCHUNK = P // CHUNK
UNROLL = 2
ROWS_PER_BLK = 8
MASKL = jnp.int32(0xFFFF)
MASKH = jnp.int32(-0x10000)


def _pack_img_body(v_ref, out_ref):
    v = v_ref[...]
    a = lax.bitcast_convert_type(
        v.astype(jnp.bfloat16), jnp.uint16).astype(jnp.int32)
    nxt = jnp.roll(a, -1, axis=1)
    out_ref[...] = a | (nxt << 16)


def _pack_img(img):
    v = img.reshape(-1, P)
    return pl.pallas_call(
        _pack_img_body,
        grid=(PLANES // ROWS_PER_BLK,),
        in_specs=[pl.BlockSpec((ROWS_PER_BLK, P), lambda i: (i, 0))],
        out_specs=pl.BlockSpec((ROWS_PER_BLK, P), lambda i: (i, 0)),
        out_shape=jax.ShapeDtypeStruct((PLANES, P), jnp.int32),
    )(v).reshape(PLANES * P)


def _coef_body(xs_ref, ys_ref, pk):
    xs = xs_ref[...]
    ys = ys_ref[...]
    x = (xs + 1.0) * jnp.float32(W) / 2.0
    y = (ys + 1.0) * jnp.float32(H) / 2.0
    x0r = jnp.floor(x).astype(jnp.int32)
    y0r = jnp.floor(y).astype(jnp.int32)
    x0 = jnp.clip(x0r, 0, W - 1)
    x1 = jnp.clip(x0r + 1, 0, W - 1)
    y0 = jnp.clip(y0r, 0, H - 1)
    y1 = jnp.clip(y0r + 1, 0, H - 1)
    x0f = x0.astype(jnp.float32)
    x1f = x1.astype(jnp.float32)
    y0f = y0.astype(jnp.float32)
    y1f = y1.astype(jnp.float32)
    zero = jnp.float32(0.0)
    gx0 = jnp.where(x1 > x0, x1f - x, zero)
    gx1 = jnp.where(x1 > x0, x - x0f, zero)
    gy0 = jnp.where(y1 > y0, y1f - y, zero)
    gy1 = jnp.where(y1 > y0, y - y0f, zero)
    ia = y0 * W + x0
    ib = y1 * W + x0

    def b16(v):
        return lax.bitcast_convert_type(
            v.astype(jnp.bfloat16), jnp.uint16).astype(jnp.int32)

    pk[:, :, 0, :] = ia | (ib << 16)
    pk[:, :, 1, :] = (b16(gx0) << 16) | b16(gx1)
    pk[:, :, 2, :] = (b16(gy0) << 16) | b16(gy1)


def _coefs(theta):
    x_lin = jnp.linspace(-1.0, 1.0, W, dtype=jnp.float32)
    y_lin = jnp.linspace(-1.0, 1.0, H, dtype=jnp.float32)
    x_t = jnp.tile(x_lin[None, :], (H, 1))
    y_t = jnp.tile(y_lin[:, None], (1, W))
    grid = jnp.stack(
        [x_t.ravel(), y_t.ravel(), jnp.ones(P, dtype=jnp.float32)], axis=0)
    theta_r = theta.reshape(-1, 2, 3)
    t_g = jnp.einsum('bij,jp->bip', theta_r, grid)
    xs = t_g[:, 0, :].reshape(B, NCHUNK, CHUNK)
    ys = t_g[:, 1, :].reshape(B, NCHUNK, CHUNK)
    return pl.pallas_call(
        _coef_body,
        out_shape=jax.ShapeDtypeStruct((B, NCHUNK, 3, CHUNK), jnp.int32),
    )(xs, ys)


def _sc_body(img_hbm, pk_hbm, out_hbm,
             img0, img1, pk0, pk1, oa0, oa1, ob0, ob1,
             sem0, sem1, semw0, semw1):
    wid = lax.axis_index("s") * NC + lax.axis_index("c")
    base_g = wid * PLANES_PER_TILE
    b = base_g // C

    def fire(k, buf, sem):
        src = pk_hbm.at[pl.ds((b * NCHUNK + k) * (3 * CHUNK), 3 * CHUNK)]
        pltpu.async_copy(src, buf, sem)

    def drain(buf, sem):
        pltpu.make_async_copy(
            pk_hbm.at[pl.ds(0, 3 * CHUNK)], buf, sem).wait()

    def compute(buf, outa, outb):
        @plsc.parallel_loop(0, CHUNK, step=L, unroll=UNROLL)
        def _body(o):
            w1 = buf[pl.ds(0 * CHUNK + o, L)]
            gxp = buf[pl.ds(1 * CHUNK + o, L)]
            gyp = buf[pl.ds(2 * CHUNK + o, L)]
            ia = w1 & MASKL
            ib = lax.shift_right_logical(w1, 16)
            gx0 = plsc.bitcast(gxp & MASKH, jnp.float32)
            gx1 = plsc.bitcast(gxp << 16, jnp.float32)
            gy0 = plsc.bitcast(gyp & MASKH, jnp.float32)
            gy1 = plsc.bitcast(gyp << 16, jnp.float32)
            for imgv, outv in ((img0, outa), (img1, outb)):
                w0 = plsc.load_gather(imgv, [ia])
                w2 = plsc.load_gather(imgv, [ib])
                va = plsc.bitcast(w0 << 16, jnp.float32)
                vc = plsc.bitcast(w0 & MASKH, jnp.float32)
                vb = plsc.bitcast(w2 << 16, jnp.float32)
                vd = plsc.bitcast(w2 & MASKH, jnp.float32)
                outv[pl.ds(o, L)] = (gy0 * (gx0 * va + gx1 * vc)
                                     + gy1 * (gx0 * vb + gx1 * vd))

    def wout(g, k, outa, outb, semw):
        pltpu.async_copy(
            outa, out_hbm.at[pl.ds(g * P + k * CHUNK, CHUNK)], semw)
        pltpu.async_copy(
            outb, out_hbm.at[pl.ds((g + 1) * P + k * CHUNK, CHUNK)], semw)

    def wdrain(outa, outb, semw):
        pltpu.make_async_copy(
            out_hbm.at[pl.ds(0, CHUNK)], outa, semw).wait()
        pltpu.make_async_copy(
            out_hbm.at[pl.ds(0, CHUNK)], outb, semw).wait()

    def per_pair(pr, carry):
        g = base_g + 2 * pr
        pltpu.sync_copy(img_hbm.at[pl.ds(g * P, P)], img0)
        pltpu.sync_copy(img_hbm.at[pl.ds((g + 1) * P, P)], img1)
        fire(0, pk0, sem0)

        def pair(k2, carry2):
            k = 2 * k2
            fire(k + 1, pk1, sem1)
            drain(pk0, sem0)

            @pl.when(k2 > 0)
            def _():
                wdrain(oa0, ob0, semw0)

            compute(pk0, oa0, ob0)
            wout(g, k, oa0, ob0, semw0)

            @pl.when(k2 < NCHUNK // 2 - 1)
            def _():
                fire(k + 2, pk0, sem0)

            drain(pk1, sem1)

            @pl.when(k2 > 0)
            def _():
                wdrain(oa1, ob1, semw1)

            compute(pk1, oa1, ob1)
            wout(g, k + 1, oa1, ob1, semw1)
            return carry2

        lax.fori_loop(0, NCHUNK // 2, pair, jnp.int32(0))
        wdrain(oa0, ob0, semw0)
        wdrain(oa1, ob1, semw1)
        return carry

    lax.fori_loop(0, PAIRS_PER_TILE, per_pair, jnp.int32(0))


@functools.cache
def _sc_sample_call():
    return pl.kernel(
        _sc_body,
        out_type=jax.ShapeDtypeStruct((PLANES * P,), jnp.float32),
        mesh=plsc.VectorSubcoreMesh(
            core_axis_name="c", subcore_axis_name="s",
            num_cores=NC, num_subcores=NS),
        compiler_params=pltpu.CompilerParams(needs_layout_passes=False),
        scratch_types=[
            pltpu.VMEM((P,), jnp.int32),
            pltpu.VMEM((P,), jnp.int32),
            pltpu.VMEM((3 * CHUNK,), jnp.int32),
            pltpu.VMEM((3 * CHUNK,), jnp.int32),
            pltpu.VMEM((CHUNK,), jnp.float32),
            pltpu.VMEM((CHUNK,), jnp.float32),
            pltpu.VMEM((CHUNK,), jnp.float32),
            pltpu.VMEM((CHUNK,), jnp.float32),
            pltpu.SemaphoreType.DMA,
            pltpu.SemaphoreType.DMA,
            pltpu.SemaphoreType.DMA,
            pltpu.SemaphoreType.DMA,
        ],
    )


def kernel(conv_input, theta):
    pk = _coefs(theta).reshape(-1)
    pimg = _pack_img(conv_input.reshape(PLANES * P))
    out = _sc_sample_call()(pimg, pk)
    return out.reshape(B, C, H, W)


# speedup vs baseline: 1.2197x; 1.0012x over previous
import functools

import jax
import jax.numpy as jnp
from jax import lax
from jax.experimental import pallas as pl
from jax.experimental.pallas import tpu as pltpu
from jax.experimental.pallas import tpu_sc as plsc

B, C, H, W = 4, 192, 224, 224
P = H * W
NC, NS, L = 2, 16, 16
NW = NC * NS
PLANES = B * C
PLANES_PER_TILE = PLANES // NW
PAIRS_PER_TILE = PLANES_PER_TILE // 2
CHUNK = 1792
NCHUNK = P // CHUNK
UNROLL = 4
ROWS_PER_BLK = 8
MASKL = jnp.int32(0xFFFF)
MASKH = jnp.int32(-0x10000)


def _pack_img_body(v_ref, out_ref):
    v = v_ref[...]
    a = lax.bitcast_convert_type(
        v.astype(jnp.bfloat16), jnp.uint16).astype(jnp.int32)
    nxt = jnp.roll(a, -1, axis=1)
    out_ref[...] = a | (nxt << 16)


def _pack_img(img):
    v = img.reshape(-1, P)
    return pl.pallas_call(
        _pack_img_body,
        grid=(PLANES // ROWS_PER_BLK,),
        in_specs=[pl.BlockSpec((ROWS_PER_BLK, P), lambda i: (i, 0))],
        out_specs=pl.BlockSpec((ROWS_PER_BLK, P), lambda i: (i, 0)),
        out_shape=jax.ShapeDtypeStruct((PLANES, P), jnp.int32),
    )(v).reshape(PLANES * P)


def _coef_body(xs_ref, ys_ref, pk):
    xs = xs_ref[...]
    ys = ys_ref[...]
    x = (xs + 1.0) * jnp.float32(W) / 2.0
    y = (ys + 1.0) * jnp.float32(H) / 2.0
    x0r = jnp.floor(x).astype(jnp.int32)
    y0r = jnp.floor(y).astype(jnp.int32)
    x0 = jnp.clip(x0r, 0, W - 1)
    x1 = jnp.clip(x0r + 1, 0, W - 1)
    y0 = jnp.clip(y0r, 0, H - 1)
    y1 = jnp.clip(y0r + 1, 0, H - 1)
    x0f = x0.astype(jnp.float32)
    x1f = x1.astype(jnp.float32)
    y0f = y0.astype(jnp.float32)
    y1f = y1.astype(jnp.float32)
    zero = jnp.float32(0.0)
    gx0 = jnp.where(x1 > x0, x1f - x, zero)
    gx1 = jnp.where(x1 > x0, x - x0f, zero)
    gy0 = jnp.where(y1 > y0, y1f - y, zero)
    gy1 = jnp.where(y1 > y0, y - y0f, zero)
    ia = y0 * W + x0
    ib = y1 * W + x0

    def b16(v):
        return lax.bitcast_convert_type(
            v.astype(jnp.bfloat16), jnp.uint16).astype(jnp.int32)

    pk[:, :, 0, :] = ia | (ib << 16)
    pk[:, :, 1, :] = (b16(gx0) << 16) | b16(gx1)
    pk[:, :, 2, :] = (b16(gy0) << 16) | b16(gy1)


def _coefs(theta):
    x_lin = jnp.linspace(-1.0, 1.0, W, dtype=jnp.float32)
    y_lin = jnp.linspace(-1.0, 1.0, H, dtype=jnp.float32)
    x_t = jnp.tile(x_lin[None, :], (H, 1))
    y_t = jnp.tile(y_lin[:, None], (1, W))
    grid = jnp.stack(
        [x_t.ravel(), y_t.ravel(), jnp.ones(P, dtype=jnp.float32)], axis=0)
    theta_r = theta.reshape(-1, 2, 3)
    t_g = jnp.einsum('bij,jp->bip', theta_r, grid)
    xs = t_g[:, 0, :].reshape(B, NCHUNK, CHUNK)
    ys = t_g[:, 1, :].reshape(B, NCHUNK, CHUNK)
    return pl.pallas_call(
        _coef_body,
        out_shape=jax.ShapeDtypeStruct((B, NCHUNK, 3, CHUNK), jnp.int32),
    )(xs, ys)


def _sc_body(img_hbm, pk_hbm, out_hbm,
             img0, img1, pk0, pk1, oa0, oa1, ob0, ob1,
             sem0, sem1, semw0, semw1):
    wid = lax.axis_index("s") * NC + lax.axis_index("c")
    base_g = wid * PLANES_PER_TILE
    b = base_g // C

    def fire(k, buf, sem):
        src = pk_hbm.at[pl.ds((b * NCHUNK + k) * (3 * CHUNK), 3 * CHUNK)]
        pltpu.async_copy(src, buf, sem)

    def drain(buf, sem):
        pltpu.make_async_copy(
            pk_hbm.at[pl.ds(0, 3 * CHUNK)], buf, sem).wait()

    def compute(buf, outa, outb):
        @plsc.parallel_loop(0, CHUNK, step=L, unroll=UNROLL)
        def _body(o):
            w1 = buf[pl.ds(0 * CHUNK + o, L)]
            gxp = buf[pl.ds(1 * CHUNK + o, L)]
            gyp = buf[pl.ds(2 * CHUNK + o, L)]
            ia = w1 & MASKL
            ib = lax.shift_right_logical(w1, 16)
            gx0 = plsc.bitcast(gxp & MASKH, jnp.float32)
            gx1 = plsc.bitcast(gxp << 16, jnp.float32)
            gy0 = plsc.bitcast(gyp & MASKH, jnp.float32)
            gy1 = plsc.bitcast(gyp << 16, jnp.float32)
            for imgv, outv in ((img0, outa), (img1, outb)):
                w0 = plsc.load_gather(imgv, [ia])
                w2 = plsc.load_gather(imgv, [ib])
                va = plsc.bitcast(w0 << 16, jnp.float32)
                vc = plsc.bitcast(w0 & MASKH, jnp.float32)
                vb = plsc.bitcast(w2 << 16, jnp.float32)
                vd = plsc.bitcast(w2 & MASKH, jnp.float32)
                outv[pl.ds(o, L)] = (gy0 * (gx0 * va + gx1 * vc)
                                     + gy1 * (gx0 * vb + gx1 * vd))

    def wout(g, k, outa, outb, semw):
        pltpu.async_copy(
            outa, out_hbm.at[pl.ds(g * P + k * CHUNK, CHUNK)], semw)
        pltpu.async_copy(
            outb, out_hbm.at[pl.ds((g + 1) * P + k * CHUNK, CHUNK)], semw)

    def wdrain(outa, outb, semw):
        pltpu.make_async_copy(
            out_hbm.at[pl.ds(0, CHUNK)], outa, semw).wait()
        pltpu.make_async_copy(
            out_hbm.at[pl.ds(0, CHUNK)], outb, semw).wait()

    def per_pair(pr, carry):
        g = base_g + 2 * pr
        pltpu.sync_copy(img_hbm.at[pl.ds(g * P, P)], img0)
        pltpu.sync_copy(img_hbm.at[pl.ds((g + 1) * P, P)], img1)
        fire(0, pk0, sem0)

        def pair(k2, carry2):
            k = 2 * k2
            fire(k + 1, pk1, sem1)
            drain(pk0, sem0)

            @pl.when(k2 > 0)
            def _():
                wdrain(oa0, ob0, semw0)

            compute(pk0, oa0, ob0)
            wout(g, k, oa0, ob0, semw0)

            @pl.when(k2 < NCHUNK // 2 - 1)
            def _():
                fire(k + 2, pk0, sem0)

            drain(pk1, sem1)

            @pl.when(k2 > 0)
            def _():
                wdrain(oa1, ob1, semw1)

            compute(pk1, oa1, ob1)
            wout(g, k + 1, oa1, ob1, semw1)
            return carry2

        lax.fori_loop(0, NCHUNK // 2, pair, jnp.int32(0))
        wdrain(oa0, ob0, semw0)
        wdrain(oa1, ob1, semw1)
        return carry

    lax.fori_loop(0, PAIRS_PER_TILE, per_pair, jnp.int32(0))


@functools.cache
def _sc_sample_call():
    return pl.kernel(
        _sc_body,
        out_type=jax.ShapeDtypeStruct((PLANES * P,), jnp.float32),
        mesh=plsc.VectorSubcoreMesh(
            core_axis_name="c", subcore_axis_name="s",
            num_cores=NC, num_subcores=NS),
        compiler_params=pltpu.CompilerParams(needs_layout_passes=False),
        scratch_types=[
            pltpu.VMEM((P,), jnp.int32),
            pltpu.VMEM((P,), jnp.int32),
            pltpu.VMEM((3 * CHUNK,), jnp.int32),
            pltpu.VMEM((3 * CHUNK,), jnp.int32),
            pltpu.VMEM((CHUNK,), jnp.float32),
            pltpu.VMEM((CHUNK,), jnp.float32),
            pltpu.VMEM((CHUNK,), jnp.float32),
            pltpu.VMEM((CHUNK,), jnp.float32),
            pltpu.SemaphoreType.DMA,
            pltpu.SemaphoreType.DMA,
            pltpu.SemaphoreType.DMA,
            pltpu.SemaphoreType.DMA,
        ],
    )


def kernel(conv_input, theta):
    pk = _coefs(theta).reshape(-1)
    pimg = _pack_img(conv_input.reshape(PLANES * P))
    out = _sc_sample_call()(pimg, pk)
    return out.reshape(B, C, H, W)
